# baseline (speedup 1.0000x reference)
import functools

import jax
import jax.numpy as jnp
from jax import lax
from jax.experimental import pallas as pl
from jax.experimental.pallas import tpu as pltpu
from jax.experimental.pallas import tpu_sc as plsc

_N, _DIN, _D, _E, _H, _K, _C = 2048, 2048, 1024, 8, 2048, 2, 1000
_T = 64
_G = (_N * _K) // _T + _E
_SIZE = _G * _T
_BN = 256
_NB = _N // _BN
_PB = 512
_MB = _N // _PB
_NW = 32
_PPW = (_N * _K) // _NW
_CH = 32


def _trunk_body(x_ref, wt_ref, bt_ref, wg_ref, bg_ref, feat_ref, p_ref):
    f = jnp.dot(x_ref[...], wt_ref[...], preferred_element_type=jnp.float32)
    f = jax.nn.gelu(f + bt_ref[...])
    feat_ref[...] = f
    gl = jnp.dot(f, wg_ref[...], preferred_element_type=jnp.float32) + bg_ref[...]
    p_ref[...] = jax.nn.softmax(gl, axis=-1)


def _trunk_gate(x, W_trunk, b_trunk, Wg, bg):
    return pl.pallas_call(
        _trunk_body,
        grid=(_NB,),
        in_specs=[
            pl.BlockSpec((_BN, _DIN), lambda i: (i, 0)),
            pl.BlockSpec((_DIN, _D), lambda i: (0, 0)),
            pl.BlockSpec((1, _D), lambda i: (0, 0)),
            pl.BlockSpec((_D, _E), lambda i: (0, 0)),
            pl.BlockSpec((1, _E), lambda i: (0, 0)),
        ],
        out_specs=[
            pl.BlockSpec((_BN, _D), lambda i: (i, 0)),
            pl.BlockSpec((_BN, _E), lambda i: (i, 0)),
        ],
        out_shape=[
            jax.ShapeDtypeStruct((_N, _D), jnp.float32),
            jax.ShapeDtypeStruct((_N, _E), jnp.float32),
        ],
    )(x, W_trunk, b_trunk.reshape(1, _D), Wg, bg.reshape(1, _E))


def _top2(pblk):
    lane = lax.broadcasted_iota(jnp.int32, pblk.shape, 1)
    t1 = jnp.max(pblk, axis=-1, keepdims=True)
    i1 = jnp.min(jnp.where(pblk == t1, lane, _E), axis=-1)
    pm = jnp.where(lane == i1[:, None], -1.0, pblk)
    t2 = jnp.max(pm, axis=-1, keepdims=True)
    i2 = jnp.min(jnp.where(pm == t2, lane, _E), axis=-1)
    return t1, i1, t2, i2


def _onehot(ids):
    return (ids[:, None] == lax.broadcasted_iota(jnp.int32, (ids.shape[0], _E), 1)
            ).astype(jnp.float32)


def _meta_body(p_ref, dest_ref, eid_ref, cnt, runoff):
    ph = pl.program_id(0)
    b = pl.program_id(1)
    pblk = p_ref[...]
    _, i1, _, i2 = _top2(pblk)
    oh1, oh2 = _onehot(i1), _onehot(i2)

    @pl.when((ph == 0) & (b == 0))
    def _():
        cnt[...] = jnp.zeros_like(cnt)

    @pl.when(ph == 0)
    def _():
        cnt[...] += (jnp.sum(oh1, axis=0, keepdims=True)
                     + jnp.sum(oh2, axis=0, keepdims=True))
        dest_ref[0, 0, 0, 0, :] = jnp.zeros((_PB,), jnp.int32)
        dest_ref[0, 1, 0, 0, :] = jnp.zeros((_PB,), jnp.int32)

    @pl.when((ph == 1) & (b == 0))
    def _():
        c = cnt[...]
        padc = jnp.ceil(c * (1.0 / _T)) * _T
        tri = (lax.broadcasted_iota(jnp.int32, (_E, _E), 0)
               < lax.broadcasted_iota(jnp.int32, (_E, _E), 1)).astype(jnp.float32)
        off = jnp.dot(padc, tri, preferred_element_type=jnp.float32)
        runoff[0:1, :] = off
        runoff[1:2, :] = jnp.zeros((1, _E), jnp.float32)
        endt = (off + padc) * (1.0 / _T)
        tid = lax.broadcasted_iota(jnp.int32, (1, 128), 1).astype(jnp.float32)
        acc = jnp.zeros((1, 128), jnp.float32)
        for e in range(_E):
            acc += (tid >= endt[:, e:e + 1]).astype(jnp.float32)
        eid = jnp.minimum(acc, _E - 1)
        used = jnp.broadcast_to(endt[:, _E - 1:_E], (1, 128))
        lanes = lax.broadcasted_iota(jnp.int32, (1, 128), 1)
        eid_ref[...] = jnp.where(lanes == _G, used, eid).astype(jnp.int32)

    @pl.when(ph == 1)
    def _():
        tri = (lax.broadcasted_iota(jnp.int32, (_PB, _PB), 0)
               >= lax.broadcasted_iota(jnp.int32, (_PB, _PB), 1)).astype(jnp.float32)
        off = runoff[0:1, :]
        run = runoff[1:2, :]
        cum1 = jnp.dot(tri, oh1, preferred_element_type=jnp.float32)
        dest1 = jnp.sum(oh1 * (off + run + cum1 - 1.0), axis=-1)
        run = run + jnp.sum(oh1, axis=0, keepdims=True)
        cum2 = jnp.dot(tri, oh2, preferred_element_type=jnp.float32)
        dest2 = jnp.sum(oh2 * (off + run + cum2 - 1.0), axis=-1)
        run = run + jnp.sum(oh2, axis=0, keepdims=True)
        runoff[1:2, :] = run
        dest_ref[0, 0, 0, 0, :] = dest1.astype(jnp.int32)
        dest_ref[0, 1, 0, 0, :] = dest2.astype(jnp.int32)


def _metadata(p):
    return pl.pallas_call(
        _meta_body,
        grid=(2, _MB),
        in_specs=[pl.BlockSpec((_PB, _E), lambda ph, b: (b, 0))],
        out_specs=[
            pl.BlockSpec((1, _K, 1, 1, _PB), lambda ph, b: (ph, 0, b, 0, 0)),
            pl.BlockSpec((1, 128), lambda ph, b: (0, 0)),
        ],
        out_shape=[
            jax.ShapeDtypeStruct((2, _K, _MB, 1, _PB), jnp.int32),
            jax.ShapeDtypeStruct((1, 128), jnp.int32),
        ],
        scratch_shapes=[
            pltpu.VMEM((1, _E), jnp.float32),
            pltpu.VMEM((2, _E), jnp.float32),
        ],
    )(p)


_NCH = _PPW // _CH


def _dispatch_sc(feat, dest2d):
    mesh = plsc.VectorSubcoreMesh(core_axis_name="c", subcore_axis_name="s")

    @functools.partial(
        pl.kernel, mesh=mesh,
        out_type=jax.ShapeDtypeStruct((_SIZE, _D), jnp.float32),
        scratch_types=[
            pltpu.VMEM((_NCH, _CH), jnp.int32),
            pltpu.VMEM((_CH, _D), jnp.float32),
            pltpu.VMEM((_CH, _D), jnp.float32),
            pltpu.SemaphoreType.DMA,
            pltpu.SemaphoreType.DMA,
            pltpu.SemaphoreType.DMA,
            pltpu.SemaphoreType.DMA,
        ],
    )
    def k(feat_hbm, dest_hbm, out_hbm, dst_v, buf0, buf1, g0, g1, s0, s1):
        wid = lax.axis_index("s") * 2 + lax.axis_index("c")
        tok0 = lax.rem(wid * _PPW, _N)
        bufs, gsem, ssem = (buf0, buf1), (g0, g1), (s0, s1)
        pltpu.sync_copy(dest_hbm.at[pl.ds(wid * _NCH, _NCH)], dst_v)
        rd = {0: pltpu.async_copy(
            feat_hbm.at[pl.ds(tok0, _CH)], bufs[0], gsem[0])}
        wr = {}
        for it in range(_NCH):
            b = it % 2
            rd[it].wait()
            if it >= 1:
                wr[it - 1].wait()
            if it + 1 < _NCH:
                nb = (it + 1) % 2
                rd[it + 1] = pltpu.async_copy(
                    feat_hbm.at[pl.ds(tok0 + (it + 1) * _CH, _CH)],
                    bufs[nb], gsem[nb])
            wr[it] = pltpu.async_copy(bufs[b], out_hbm.at[dst_v.at[it]],
                                      ssem[b])
        wr[_NCH - 1].wait()

    return k(feat, dest2d)


def _mlp_body(eid_ref, x_ref, w1_ref, b1_ref, w2_ref, b2_ref, o_ref):
    @pl.when(pl.program_id(0) < eid_ref[_G])
    def _():
        h = jnp.dot(x_ref[...], w1_ref[0], preferred_element_type=jnp.float32)
        h = jnp.maximum(h + b1_ref[0], 0.0)
        o = jnp.dot(h, w2_ref[0], preferred_element_type=jnp.float32)
        o_ref[...] = o + b2_ref[0]


def _grouped_mlp(tile_eid, feat_sorted, W1, b1, W2, b2):
    grid_spec = pltpu.PrefetchScalarGridSpec(
        num_scalar_prefetch=1,
        grid=(_G,),
        in_specs=[
            pl.BlockSpec((_T, _D), lambda t, eid: (t, 0)),
            pl.BlockSpec((1, _D, _H), lambda t, eid: (eid[t], 0, 0)),
            pl.BlockSpec((1, 1, _H), lambda t, eid: (eid[t], 0, 0)),
            pl.BlockSpec((1, _H, _D), lambda t, eid: (eid[t], 0, 0)),
            pl.BlockSpec((1, 1, _D), lambda t, eid: (eid[t], 0, 0)),
        ],
        out_specs=pl.BlockSpec((_T, _D), lambda t, eid: (t, 0)),
    )
    return pl.pallas_call(
        _mlp_body,
        grid_spec=grid_spec,
        out_shape=jax.ShapeDtypeStruct((_SIZE, _D), jnp.float32),
    )(tile_eid, feat_sorted, W1, b1.reshape(_E, 1, _H), W2, b2.reshape(_E, 1, _D))


_PPW2 = _N // _NW
_NCH2 = _PPW2 // _CH


def _gather_half_sc(table, idx_flat, h):
    mesh = plsc.VectorSubcoreMesh(core_axis_name="c", subcore_axis_name="s")

    @functools.partial(
        pl.kernel, mesh=mesh,
        out_type=jax.ShapeDtypeStruct((_N, _D), jnp.float32),
        scratch_types=[
            pltpu.VMEM((_PPW2,), jnp.int32),
            pltpu.VMEM((_CH, _D), jnp.float32),
            pltpu.VMEM((_CH, _D), jnp.float32),
            pltpu.SemaphoreType.DMA,
            pltpu.SemaphoreType.DMA,
            pltpu.SemaphoreType.DMA,
            pltpu.SemaphoreType.DMA,
        ],
    )
    def k(table_hbm, idx_hbm, out_hbm, idx_v, buf0, buf1, g0, g1, s0, s1):
        wid = lax.axis_index("s") * 2 + lax.axis_index("c")
        lbase = wid * _PPW2
        gbase = jnp.where(wid < _NW // 2,
                          h * (_N // 2) + wid * _PPW2,
                          _N + h * (_N // 2) + (wid - _NW // 2) * _PPW2)
        bufs, gsem, ssem = (buf0, buf1), (g0, g1), (s0, s1)
        pltpu.sync_copy(idx_hbm.at[pl.ds(gbase, _PPW2)], idx_v)
        rd = {0: pltpu.async_copy(
            table_hbm.at[idx_v.at[pl.ds(0, _CH)]], bufs[0], gsem[0])}
        wr = {}
        for it in range(_NCH2):
            b = it % 2
            rd[it].wait()
            if it >= 1:
                wr[it - 1].wait()
            if it + 1 < _NCH2:
                nb = (it + 1) % 2
                rd[it + 1] = pltpu.async_copy(
                    table_hbm.at[idx_v.at[pl.ds((it + 1) * _CH, _CH)]],
                    bufs[nb], gsem[nb])
            wr[it] = pltpu.async_copy(bufs[b],
                                      out_hbm.at[pl.ds(lbase + it * _CH, _CH)],
                                      ssem[b])
        wr[_NCH2 - 1].wait()

    return k(table, idx_flat)


def _combine_body(r0_ref, r1_ref, p_ref, wc_ref, bc_ref, o_ref):
    t1, _, t2, _ = _top2(p_ref[...])
    s = t1 + t2
    moe = (t1 / s) * r0_ref[...] + (t2 / s) * r1_ref[...]
    o = jnp.dot(moe, wc_ref[...], preferred_element_type=jnp.float32)
    o_ref[...] = o + bc_ref[...]


def _combine_body2(r0_ref, r1_ref, p_ref, wc_ref, bc_ref, prev_ref, o_ref):
    _combine_body(r0_ref, r1_ref, p_ref, wc_ref, bc_ref, o_ref)


def _combine_half(rows01, p_half, Wc, bc, prev, h):
    nb = (_N // 2) // _BN
    in_specs = [
        pl.BlockSpec((_BN, _D), lambda i: (i, 0)),
        pl.BlockSpec((_BN, _D), lambda i: (i + nb, 0)),
        pl.BlockSpec((_BN, _E), lambda i: (i, 0)),
        pl.BlockSpec((_D, _C), lambda i: (0, 0)),
        pl.BlockSpec((1, _C), lambda i: (0, 0)),
    ]
    args = [rows01, rows01, p_half, Wc, bc.reshape(1, _C)]
    body, aliases = _combine_body, {}
    if prev is not None:
        in_specs.append(pl.BlockSpec(memory_space=pl.ANY))
        args.append(prev)
        body, aliases = _combine_body2, {5: 0}
    return pl.pallas_call(
        body,
        grid=(nb,),
        in_specs=in_specs,
        out_specs=pl.BlockSpec((_BN, _C), lambda i, _h=h: (i + _h * nb, 0)),
        out_shape=jax.ShapeDtypeStruct((_N, _C), jnp.float32),
        input_output_aliases=aliases,
    )(*args)


def kernel(x, W_trunk, b_trunk, Wg, bg, W1, b1, W2, b2, Wc, bc):
    feat, p = _trunk_gate(x, W_trunk, b_trunk, Wg, bg)
    dest5, eid2 = _metadata(p)
    dest_flat = dest5[1].reshape(_N * _K)
    tile_eid = eid2[0, :_G + 1]
    feat_sorted = _dispatch_sc(feat, dest_flat.reshape(_NW * _NCH, _CH))
    out_sorted = _grouped_mlp(tile_eid, feat_sorted, W1, b1, W2, b2)
    g0 = _gather_half_sc(out_sorted, dest_flat, 0)
    g1 = _gather_half_sc(out_sorted, dest_flat, 1)
    logits0 = _combine_half(g0, p[:_N // 2], Wc, bc, None, 0)
    logits = _combine_half(g1, p[_N // 2:], Wc, bc, logits0, 1)
    return logits, p


# speedup vs baseline: 1.3122x; 1.3122x over previous
import functools

import jax
import jax.numpy as jnp
from jax import lax
from jax.experimental import pallas as pl
from jax.experimental.pallas import tpu as pltpu
from jax.experimental.pallas import tpu_sc as plsc

_N, _DIN, _D, _E, _H, _K, _C = 2048, 2048, 1024, 8, 2048, 2, 1000
_T = 256
_G = (_N * _K) // _T + _E
_SIZE = _G * _T
_BN = 256
_NB = _N // _BN
_PB = 512
_MB = _N // _PB
_NW = 32
_PPW = (_N * _K) // _NW
_CH = 32


def _trunk_body(x_ref, wt_ref, bt_ref, wg_ref, bg_ref, feat_ref, p_ref):
    f = jnp.dot(x_ref[...], wt_ref[...], preferred_element_type=jnp.float32)
    f = jax.nn.gelu(f + bt_ref[...])
    feat_ref[...] = f
    gl = jnp.dot(f, wg_ref[...], preferred_element_type=jnp.float32) + bg_ref[...]
    p_ref[...] = jax.nn.softmax(gl, axis=-1)


def _trunk_gate(x, W_trunk, b_trunk, Wg, bg):
    return pl.pallas_call(
        _trunk_body,
        grid=(_NB,),
        in_specs=[
            pl.BlockSpec((_BN, _DIN), lambda i: (i, 0)),
            pl.BlockSpec((_DIN, _D), lambda i: (0, 0)),
            pl.BlockSpec((1, _D), lambda i: (0, 0)),
            pl.BlockSpec((_D, _E), lambda i: (0, 0)),
            pl.BlockSpec((1, _E), lambda i: (0, 0)),
        ],
        out_specs=[
            pl.BlockSpec((_BN, _D), lambda i: (i, 0)),
            pl.BlockSpec((_BN, _E), lambda i: (i, 0)),
        ],
        out_shape=[
            jax.ShapeDtypeStruct((_N, _D), jnp.float32),
            jax.ShapeDtypeStruct((_N, _E), jnp.float32),
        ],
    )(x, W_trunk, b_trunk.reshape(1, _D), Wg, bg.reshape(1, _E))


def _top2(pblk):
    lane = lax.broadcasted_iota(jnp.int32, pblk.shape, 1)
    t1 = jnp.max(pblk, axis=-1, keepdims=True)
    i1 = jnp.min(jnp.where(pblk == t1, lane, _E), axis=-1)
    pm = jnp.where(lane == i1[:, None], -1.0, pblk)
    t2 = jnp.max(pm, axis=-1, keepdims=True)
    i2 = jnp.min(jnp.where(pm == t2, lane, _E), axis=-1)
    return t1, i1, t2, i2


def _onehot(ids):
    return (ids[:, None] == lax.broadcasted_iota(jnp.int32, (ids.shape[0], _E), 1)
            ).astype(jnp.float32)


def _meta_body(p_ref, dest_ref, eid_ref, cnt, runoff):
    ph = pl.program_id(0)
    b = pl.program_id(1)
    pblk = p_ref[...]
    _, i1, _, i2 = _top2(pblk)
    oh1, oh2 = _onehot(i1), _onehot(i2)

    @pl.when((ph == 0) & (b == 0))
    def _():
        cnt[...] = jnp.zeros_like(cnt)

    @pl.when(ph == 0)
    def _():
        cnt[...] += (jnp.sum(oh1, axis=0, keepdims=True)
                     + jnp.sum(oh2, axis=0, keepdims=True))
        dest_ref[0, 0, 0, 0, :] = jnp.zeros((_PB,), jnp.int32)
        dest_ref[0, 1, 0, 0, :] = jnp.zeros((_PB,), jnp.int32)

    @pl.when((ph == 1) & (b == 0))
    def _():
        c = cnt[...]
        padc = jnp.ceil(c * (1.0 / _T)) * _T
        tri = (lax.broadcasted_iota(jnp.int32, (_E, _E), 0)
               < lax.broadcasted_iota(jnp.int32, (_E, _E), 1)).astype(jnp.float32)
        off = jnp.dot(padc, tri, preferred_element_type=jnp.float32)
        runoff[0:1, :] = off
        runoff[1:2, :] = jnp.zeros((1, _E), jnp.float32)
        endt = (off + padc) * (1.0 / _T)
        tid = lax.broadcasted_iota(jnp.int32, (1, 128), 1).astype(jnp.float32)
        acc = jnp.zeros((1, 128), jnp.float32)
        for e in range(_E):
            acc += (tid >= endt[:, e:e + 1]).astype(jnp.float32)
        eid = jnp.minimum(acc, _E - 1)
        used = jnp.broadcast_to(endt[:, _E - 1:_E], (1, 128))
        lanes = lax.broadcasted_iota(jnp.int32, (1, 128), 1)
        eid_ref[...] = jnp.where(lanes == _G, used, eid).astype(jnp.int32)

    @pl.when(ph == 1)
    def _():
        tri = (lax.broadcasted_iota(jnp.int32, (_PB, _PB), 0)
               >= lax.broadcasted_iota(jnp.int32, (_PB, _PB), 1)).astype(jnp.float32)
        off = runoff[0:1, :]
        run = runoff[1:2, :]
        cum1 = jnp.dot(tri, oh1, preferred_element_type=jnp.float32)
        dest1 = jnp.sum(oh1 * (off + run + cum1 - 1.0), axis=-1)
        run = run + jnp.sum(oh1, axis=0, keepdims=True)
        cum2 = jnp.dot(tri, oh2, preferred_element_type=jnp.float32)
        dest2 = jnp.sum(oh2 * (off + run + cum2 - 1.0), axis=-1)
        run = run + jnp.sum(oh2, axis=0, keepdims=True)
        runoff[1:2, :] = run
        dest_ref[0, 0, 0, 0, :] = dest1.astype(jnp.int32)
        dest_ref[0, 1, 0, 0, :] = dest2.astype(jnp.int32)


def _metadata(p):
    return pl.pallas_call(
        _meta_body,
        grid=(2, _MB),
        in_specs=[pl.BlockSpec((_PB, _E), lambda ph, b: (b, 0))],
        out_specs=[
            pl.BlockSpec((1, _K, 1, 1, _PB), lambda ph, b: (ph, 0, b, 0, 0)),
            pl.BlockSpec((1, 128), lambda ph, b: (0, 0)),
        ],
        out_shape=[
            jax.ShapeDtypeStruct((2, _K, _MB, 1, _PB), jnp.int32),
            jax.ShapeDtypeStruct((1, 128), jnp.int32),
        ],
        scratch_shapes=[
            pltpu.VMEM((1, _E), jnp.float32),
            pltpu.VMEM((2, _E), jnp.float32),
        ],
    )(p)


_NCH = _PPW // _CH


def _dispatch_sc(feat, dest2d):
    mesh = plsc.VectorSubcoreMesh(core_axis_name="c", subcore_axis_name="s")

    @functools.partial(
        pl.kernel, mesh=mesh,
        out_type=jax.ShapeDtypeStruct((_SIZE, _D), jnp.float32),
        scratch_types=[
            pltpu.VMEM((_NCH, _CH), jnp.int32),
            pltpu.VMEM((_CH, _D), jnp.float32),
            pltpu.VMEM((_CH, _D), jnp.float32),
            pltpu.SemaphoreType.DMA,
            pltpu.SemaphoreType.DMA,
            pltpu.SemaphoreType.DMA,
            pltpu.SemaphoreType.DMA,
        ],
    )
    def k(feat_hbm, dest_hbm, out_hbm, dst_v, buf0, buf1, g0, g1, s0, s1):
        wid = lax.axis_index("s") * 2 + lax.axis_index("c")
        tok0 = lax.rem(wid * _PPW, _N)
        bufs, gsem, ssem = (buf0, buf1), (g0, g1), (s0, s1)
        pltpu.sync_copy(dest_hbm.at[pl.ds(wid * _NCH, _NCH)], dst_v)
        rd = {0: pltpu.async_copy(
            feat_hbm.at[pl.ds(tok0, _CH)], bufs[0], gsem[0])}
        wr = {}
        for it in range(_NCH):
            b = it % 2
            rd[it].wait()
            if it >= 1:
                wr[it - 1].wait()
            if it + 1 < _NCH:
                nb = (it + 1) % 2
                rd[it + 1] = pltpu.async_copy(
                    feat_hbm.at[pl.ds(tok0 + (it + 1) * _CH, _CH)],
                    bufs[nb], gsem[nb])
            wr[it] = pltpu.async_copy(bufs[b], out_hbm.at[dst_v.at[it]],
                                      ssem[b])
        wr[_NCH - 1].wait()

    return k(feat, dest2d)


def _mlp_body(eid_ref, x_ref, w1_ref, b1_ref, w2_ref, b2_ref, o_ref):
    @pl.when(pl.program_id(0) < eid_ref[_G])
    def _():
        h = jnp.dot(x_ref[...], w1_ref[0], preferred_element_type=jnp.float32)
        h = jnp.maximum(h + b1_ref[0], 0.0)
        o = jnp.dot(h, w2_ref[0], preferred_element_type=jnp.float32)
        o_ref[...] = o + b2_ref[0]


def _grouped_mlp(tile_eid, feat_sorted, W1, b1, W2, b2):
    grid_spec = pltpu.PrefetchScalarGridSpec(
        num_scalar_prefetch=1,
        grid=(_G,),
        in_specs=[
            pl.BlockSpec((_T, _D), lambda t, eid: (t, 0)),
            pl.BlockSpec((1, _D, _H), lambda t, eid: (eid[t], 0, 0)),
            pl.BlockSpec((1, 1, _H), lambda t, eid: (eid[t], 0, 0)),
            pl.BlockSpec((1, _H, _D), lambda t, eid: (eid[t], 0, 0)),
            pl.BlockSpec((1, 1, _D), lambda t, eid: (eid[t], 0, 0)),
        ],
        out_specs=pl.BlockSpec((_T, _D), lambda t, eid: (t, 0)),
    )
    return pl.pallas_call(
        _mlp_body,
        grid_spec=grid_spec,
        out_shape=jax.ShapeDtypeStruct((_SIZE, _D), jnp.float32),
    )(tile_eid, feat_sorted, W1, b1.reshape(_E, 1, _H), W2, b2.reshape(_E, 1, _D))


_PPW2 = _N // _NW
_NCH2 = _PPW2 // _CH


def _gather_half_sc(table, idx_flat, h):
    mesh = plsc.VectorSubcoreMesh(core_axis_name="c", subcore_axis_name="s")

    @functools.partial(
        pl.kernel, mesh=mesh,
        out_type=jax.ShapeDtypeStruct((_N, _D), jnp.float32),
        scratch_types=[
            pltpu.VMEM((_PPW2,), jnp.int32),
            pltpu.VMEM((_CH, _D), jnp.float32),
            pltpu.VMEM((_CH, _D), jnp.float32),
            pltpu.SemaphoreType.DMA,
            pltpu.SemaphoreType.DMA,
            pltpu.SemaphoreType.DMA,
            pltpu.SemaphoreType.DMA,
        ],
    )
    def k(table_hbm, idx_hbm, out_hbm, idx_v, buf0, buf1, g0, g1, s0, s1):
        wid = lax.axis_index("s") * 2 + lax.axis_index("c")
        lbase = wid * _PPW2
        gbase = jnp.where(wid < _NW // 2,
                          h * (_N // 2) + wid * _PPW2,
                          _N + h * (_N // 2) + (wid - _NW // 2) * _PPW2)
        bufs, gsem, ssem = (buf0, buf1), (g0, g1), (s0, s1)
        pltpu.sync_copy(idx_hbm.at[pl.ds(gbase, _PPW2)], idx_v)
        rd = {0: pltpu.async_copy(
            table_hbm.at[idx_v.at[pl.ds(0, _CH)]], bufs[0], gsem[0])}
        wr = {}
        for it in range(_NCH2):
            b = it % 2
            rd[it].wait()
            if it >= 1:
                wr[it - 1].wait()
            if it + 1 < _NCH2:
                nb = (it + 1) % 2
                rd[it + 1] = pltpu.async_copy(
                    table_hbm.at[idx_v.at[pl.ds((it + 1) * _CH, _CH)]],
                    bufs[nb], gsem[nb])
            wr[it] = pltpu.async_copy(bufs[b],
                                      out_hbm.at[pl.ds(lbase + it * _CH, _CH)],
                                      ssem[b])
        wr[_NCH2 - 1].wait()

    return k(table, idx_flat)


def _combine_body(r0_ref, r1_ref, p_ref, wc_ref, bc_ref, o_ref):
    t1, _, t2, _ = _top2(p_ref[...])
    s = t1 + t2
    moe = (t1 / s) * r0_ref[...] + (t2 / s) * r1_ref[...]
    o = jnp.dot(moe, wc_ref[...], preferred_element_type=jnp.float32)
    o_ref[...] = o + bc_ref[...]


def _combine_body2(r0_ref, r1_ref, p_ref, wc_ref, bc_ref, prev_ref, o_ref):
    _combine_body(r0_ref, r1_ref, p_ref, wc_ref, bc_ref, o_ref)


def _combine_half(rows01, p_half, Wc, bc, prev, h):
    nb = (_N // 2) // _BN
    in_specs = [
        pl.BlockSpec((_BN, _D), lambda i: (i, 0)),
        pl.BlockSpec((_BN, _D), lambda i: (i + nb, 0)),
        pl.BlockSpec((_BN, _E), lambda i: (i, 0)),
        pl.BlockSpec((_D, _C), lambda i: (0, 0)),
        pl.BlockSpec((1, _C), lambda i: (0, 0)),
    ]
    args = [rows01, rows01, p_half, Wc, bc.reshape(1, _C)]
    body, aliases = _combine_body, {}
    if prev is not None:
        in_specs.append(pl.BlockSpec(memory_space=pl.ANY))
        args.append(prev)
        body, aliases = _combine_body2, {5: 0}
    return pl.pallas_call(
        body,
        grid=(nb,),
        in_specs=in_specs,
        out_specs=pl.BlockSpec((_BN, _C), lambda i, _h=h: (i + _h * nb, 0)),
        out_shape=jax.ShapeDtypeStruct((_N, _C), jnp.float32),
        input_output_aliases=aliases,
    )(*args)


def kernel(x, W_trunk, b_trunk, Wg, bg, W1, b1, W2, b2, Wc, bc):
    feat, p = _trunk_gate(x, W_trunk, b_trunk, Wg, bg)
    dest5, eid2 = _metadata(p)
    dest_flat = dest5[1].reshape(_N * _K)
    tile_eid = eid2[0, :_G + 1]
    feat_sorted = _dispatch_sc(feat, dest_flat.reshape(_NW * _NCH, _CH))
    out_sorted = _grouped_mlp(tile_eid, feat_sorted, W1, b1, W2, b2)
    g0 = _gather_half_sc(out_sorted, dest_flat, 0)
    g1 = _gather_half_sc(out_sorted, dest_flat, 1)
    logits0 = _combine_half(g0, p[:_N // 2], Wc, bc, None, 0)
    logits = _combine_half(g1, p[_N // 2:], Wc, bc, logits0, 1)
    return logits, p


# speedup vs baseline: 1.3295x; 1.0132x over previous
import functools

import jax
import jax.numpy as jnp
from jax import lax
from jax.experimental import pallas as pl
from jax.experimental.pallas import tpu as pltpu
from jax.experimental.pallas import tpu_sc as plsc

_N, _DIN, _D, _E, _H, _K, _C = 2048, 2048, 1024, 8, 2048, 2, 1000
_T = 256
_G = (_N * _K) // _T + _E
_SIZE = _G * _T
_BN = 512
_NB = _N // _BN
_PB = 512
_MB = _N // _PB
_NW = 32
_PPW = (_N * _K) // _NW
_CH = 32


def _trunk_body(x_ref, wt_ref, bt_ref, wg_ref, bg_ref, feat_ref, p_ref):
    f = jnp.dot(x_ref[...], wt_ref[...], preferred_element_type=jnp.float32)
    f = jax.nn.gelu(f + bt_ref[...])
    feat_ref[...] = f
    gl = jnp.dot(f, wg_ref[...], preferred_element_type=jnp.float32) + bg_ref[...]
    p_ref[...] = jax.nn.softmax(gl, axis=-1)


def _trunk_gate(x, W_trunk, b_trunk, Wg, bg):
    return pl.pallas_call(
        _trunk_body,
        grid=(_NB,),
        in_specs=[
            pl.BlockSpec((_BN, _DIN), lambda i: (i, 0)),
            pl.BlockSpec((_DIN, _D), lambda i: (0, 0)),
            pl.BlockSpec((1, _D), lambda i: (0, 0)),
            pl.BlockSpec((_D, _E), lambda i: (0, 0)),
            pl.BlockSpec((1, _E), lambda i: (0, 0)),
        ],
        out_specs=[
            pl.BlockSpec((_BN, _D), lambda i: (i, 0)),
            pl.BlockSpec((_BN, _E), lambda i: (i, 0)),
        ],
        out_shape=[
            jax.ShapeDtypeStruct((_N, _D), jnp.float32),
            jax.ShapeDtypeStruct((_N, _E), jnp.float32),
        ],
    )(x, W_trunk, b_trunk.reshape(1, _D), Wg, bg.reshape(1, _E))


def _top2(pblk):
    lane = lax.broadcasted_iota(jnp.int32, pblk.shape, 1)
    t1 = jnp.max(pblk, axis=-1, keepdims=True)
    i1 = jnp.min(jnp.where(pblk == t1, lane, _E), axis=-1)
    pm = jnp.where(lane == i1[:, None], -1.0, pblk)
    t2 = jnp.max(pm, axis=-1, keepdims=True)
    i2 = jnp.min(jnp.where(pm == t2, lane, _E), axis=-1)
    return t1, i1, t2, i2


def _onehot(ids):
    return (ids[:, None] == lax.broadcasted_iota(jnp.int32, (ids.shape[0], _E), 1)
            ).astype(jnp.float32)


def _meta_body(p_ref, dest_ref, eid_ref, cnt, runoff):
    ph = pl.program_id(0)
    b = pl.program_id(1)
    pblk = p_ref[...]
    _, i1, _, i2 = _top2(pblk)
    oh1, oh2 = _onehot(i1), _onehot(i2)

    @pl.when((ph == 0) & (b == 0))
    def _():
        cnt[...] = jnp.zeros_like(cnt)

    @pl.when(ph == 0)
    def _():
        cnt[...] += (jnp.sum(oh1, axis=0, keepdims=True)
                     + jnp.sum(oh2, axis=0, keepdims=True))
        dest_ref[0, 0, 0, 0, :] = jnp.zeros((_PB,), jnp.int32)
        dest_ref[0, 1, 0, 0, :] = jnp.zeros((_PB,), jnp.int32)

    @pl.when((ph == 1) & (b == 0))
    def _():
        c = cnt[...]
        padc = jnp.ceil(c * (1.0 / _T)) * _T
        tri = (lax.broadcasted_iota(jnp.int32, (_E, _E), 0)
               < lax.broadcasted_iota(jnp.int32, (_E, _E), 1)).astype(jnp.float32)
        off = jnp.dot(padc, tri, preferred_element_type=jnp.float32)
        runoff[0:1, :] = off
        runoff[1:2, :] = jnp.zeros((1, _E), jnp.float32)
        endt = (off + padc) * (1.0 / _T)
        tid = lax.broadcasted_iota(jnp.int32, (1, 128), 1).astype(jnp.float32)
        acc = jnp.zeros((1, 128), jnp.float32)
        for e in range(_E):
            acc += (tid >= endt[:, e:e + 1]).astype(jnp.float32)
        eid = jnp.minimum(acc, _E - 1)
        used = jnp.broadcast_to(endt[:, _E - 1:_E], (1, 128))
        lanes = lax.broadcasted_iota(jnp.int32, (1, 128), 1)
        eid_ref[...] = jnp.where(lanes == _G, used, eid).astype(jnp.int32)

    @pl.when(ph == 1)
    def _():
        tri = (lax.broadcasted_iota(jnp.int32, (_PB, _PB), 0)
               >= lax.broadcasted_iota(jnp.int32, (_PB, _PB), 1)).astype(jnp.float32)
        off = runoff[0:1, :]
        run = runoff[1:2, :]
        cum1 = jnp.dot(tri, oh1, preferred_element_type=jnp.float32)
        dest1 = jnp.sum(oh1 * (off + run + cum1 - 1.0), axis=-1)
        run = run + jnp.sum(oh1, axis=0, keepdims=True)
        cum2 = jnp.dot(tri, oh2, preferred_element_type=jnp.float32)
        dest2 = jnp.sum(oh2 * (off + run + cum2 - 1.0), axis=-1)
        run = run + jnp.sum(oh2, axis=0, keepdims=True)
        runoff[1:2, :] = run
        dest_ref[0, 0, 0, 0, :] = dest1.astype(jnp.int32)
        dest_ref[0, 1, 0, 0, :] = dest2.astype(jnp.int32)


def _metadata(p):
    return pl.pallas_call(
        _meta_body,
        grid=(2, _MB),
        in_specs=[pl.BlockSpec((_PB, _E), lambda ph, b: (b, 0))],
        out_specs=[
            pl.BlockSpec((1, _K, 1, 1, _PB), lambda ph, b: (ph, 0, b, 0, 0)),
            pl.BlockSpec((1, 128), lambda ph, b: (0, 0)),
        ],
        out_shape=[
            jax.ShapeDtypeStruct((2, _K, _MB, 1, _PB), jnp.int32),
            jax.ShapeDtypeStruct((1, 128), jnp.int32),
        ],
        scratch_shapes=[
            pltpu.VMEM((1, _E), jnp.float32),
            pltpu.VMEM((2, _E), jnp.float32),
        ],
    )(p)


_NCH = _PPW // _CH


def _dispatch_sc(feat, dest2d):
    mesh = plsc.VectorSubcoreMesh(core_axis_name="c", subcore_axis_name="s")

    @functools.partial(
        pl.kernel, mesh=mesh,
        out_type=jax.ShapeDtypeStruct((_SIZE, _D), jnp.float32),
        scratch_types=[
            pltpu.VMEM((_NCH, _CH), jnp.int32),
            pltpu.VMEM((_CH, _D), jnp.float32),
            pltpu.VMEM((_CH, _D), jnp.float32),
            pltpu.SemaphoreType.DMA,
            pltpu.SemaphoreType.DMA,
            pltpu.SemaphoreType.DMA,
            pltpu.SemaphoreType.DMA,
        ],
    )
    def k(feat_hbm, dest_hbm, out_hbm, dst_v, buf0, buf1, g0, g1, s0, s1):
        wid = lax.axis_index("s") * 2 + lax.axis_index("c")
        tok0 = lax.rem(wid * _PPW, _N)
        bufs, gsem, ssem = (buf0, buf1), (g0, g1), (s0, s1)
        pltpu.sync_copy(dest_hbm.at[pl.ds(wid * _NCH, _NCH)], dst_v)
        rd = {0: pltpu.async_copy(
            feat_hbm.at[pl.ds(tok0, _CH)], bufs[0], gsem[0])}
        wr = {}
        for it in range(_NCH):
            b = it % 2
            rd[it].wait()
            if it >= 1:
                wr[it - 1].wait()
            if it + 1 < _NCH:
                nb = (it + 1) % 2
                rd[it + 1] = pltpu.async_copy(
                    feat_hbm.at[pl.ds(tok0 + (it + 1) * _CH, _CH)],
                    bufs[nb], gsem[nb])
            wr[it] = pltpu.async_copy(bufs[b], out_hbm.at[dst_v.at[it]],
                                      ssem[b])
        wr[_NCH - 1].wait()

    return k(feat, dest2d)


def _mlp_body(eid_ref, x_ref, w1_ref, b1_ref, w2_ref, b2_ref, o_ref):
    @pl.when(pl.program_id(0) < eid_ref[_G])
    def _():
        h = jnp.dot(x_ref[...], w1_ref[0], preferred_element_type=jnp.float32)
        h = jnp.maximum(h + b1_ref[0], 0.0)
        o = jnp.dot(h, w2_ref[0], preferred_element_type=jnp.float32)
        o_ref[...] = o + b2_ref[0]


def _grouped_mlp(tile_eid, feat_sorted, W1, b1, W2, b2):
    grid_spec = pltpu.PrefetchScalarGridSpec(
        num_scalar_prefetch=1,
        grid=(_G,),
        in_specs=[
            pl.BlockSpec((_T, _D), lambda t, eid: (t, 0)),
            pl.BlockSpec((1, _D, _H), lambda t, eid: (eid[t], 0, 0)),
            pl.BlockSpec((1, 1, _H), lambda t, eid: (eid[t], 0, 0)),
            pl.BlockSpec((1, _H, _D), lambda t, eid: (eid[t], 0, 0)),
            pl.BlockSpec((1, 1, _D), lambda t, eid: (eid[t], 0, 0)),
        ],
        out_specs=pl.BlockSpec((_T, _D), lambda t, eid: (t, 0)),
    )
    return pl.pallas_call(
        _mlp_body,
        grid_spec=grid_spec,
        out_shape=jax.ShapeDtypeStruct((_SIZE, _D), jnp.float32),
    )(tile_eid, feat_sorted, W1, b1.reshape(_E, 1, _H), W2, b2.reshape(_E, 1, _D))


_PPW2 = _N // _NW
_NCH2 = _PPW2 // _CH


def _gather_half_sc(table, idx_flat, h):
    mesh = plsc.VectorSubcoreMesh(core_axis_name="c", subcore_axis_name="s")

    @functools.partial(
        pl.kernel, mesh=mesh,
        out_type=jax.ShapeDtypeStruct((_N, _D), jnp.float32),
        scratch_types=[
            pltpu.VMEM((_PPW2,), jnp.int32),
            pltpu.VMEM((_CH, _D), jnp.float32),
            pltpu.VMEM((_CH, _D), jnp.float32),
            pltpu.SemaphoreType.DMA,
            pltpu.SemaphoreType.DMA,
            pltpu.SemaphoreType.DMA,
            pltpu.SemaphoreType.DMA,
        ],
    )
    def k(table_hbm, idx_hbm, out_hbm, idx_v, buf0, buf1, g0, g1, s0, s1):
        wid = lax.axis_index("s") * 2 + lax.axis_index("c")
        lbase = wid * _PPW2
        gbase = jnp.where(wid < _NW // 2,
                          h * (_N // 2) + wid * _PPW2,
                          _N + h * (_N // 2) + (wid - _NW // 2) * _PPW2)
        bufs, gsem, ssem = (buf0, buf1), (g0, g1), (s0, s1)
        pltpu.sync_copy(idx_hbm.at[pl.ds(gbase, _PPW2)], idx_v)
        rd = {0: pltpu.async_copy(
            table_hbm.at[idx_v.at[pl.ds(0, _CH)]], bufs[0], gsem[0])}
        wr = {}
        for it in range(_NCH2):
            b = it % 2
            rd[it].wait()
            if it >= 1:
                wr[it - 1].wait()
            if it + 1 < _NCH2:
                nb = (it + 1) % 2
                rd[it + 1] = pltpu.async_copy(
                    table_hbm.at[idx_v.at[pl.ds((it + 1) * _CH, _CH)]],
                    bufs[nb], gsem[nb])
            wr[it] = pltpu.async_copy(bufs[b],
                                      out_hbm.at[pl.ds(lbase + it * _CH, _CH)],
                                      ssem[b])
        wr[_NCH2 - 1].wait()

    return k(table, idx_flat)


def _combine_body(r0_ref, r1_ref, p_ref, wc_ref, bc_ref, o_ref):
    t1, _, t2, _ = _top2(p_ref[...])
    s = t1 + t2
    moe = (t1 / s) * r0_ref[...] + (t2 / s) * r1_ref[...]
    o = jnp.dot(moe, wc_ref[...], preferred_element_type=jnp.float32)
    o_ref[...] = o + bc_ref[...]


def _combine_body2(r0_ref, r1_ref, p_ref, wc_ref, bc_ref, prev_ref, o_ref):
    _combine_body(r0_ref, r1_ref, p_ref, wc_ref, bc_ref, o_ref)


def _combine_half(rows01, p_half, Wc, bc, prev, h):
    nb = (_N // 2) // _BN
    in_specs = [
        pl.BlockSpec((_BN, _D), lambda i: (i, 0)),
        pl.BlockSpec((_BN, _D), lambda i: (i + nb, 0)),
        pl.BlockSpec((_BN, _E), lambda i: (i, 0)),
        pl.BlockSpec((_D, _C), lambda i: (0, 0)),
        pl.BlockSpec((1, _C), lambda i: (0, 0)),
    ]
    args = [rows01, rows01, p_half, Wc, bc.reshape(1, _C)]
    body, aliases = _combine_body, {}
    if prev is not None:
        in_specs.append(pl.BlockSpec(memory_space=pl.ANY))
        args.append(prev)
        body, aliases = _combine_body2, {5: 0}
    return pl.pallas_call(
        body,
        grid=(nb,),
        in_specs=in_specs,
        out_specs=pl.BlockSpec((_BN, _C), lambda i, _h=h: (i + _h * nb, 0)),
        out_shape=jax.ShapeDtypeStruct((_N, _C), jnp.float32),
        input_output_aliases=aliases,
    )(*args)


def kernel(x, W_trunk, b_trunk, Wg, bg, W1, b1, W2, b2, Wc, bc):
    feat, p = _trunk_gate(x, W_trunk, b_trunk, Wg, bg)
    dest5, eid2 = _metadata(p)
    dest_flat = dest5[1].reshape(_N * _K)
    tile_eid = eid2[0, :_G + 1]
    feat_sorted = _dispatch_sc(feat, dest_flat.reshape(_NW * _NCH, _CH))
    out_sorted = _grouped_mlp(tile_eid, feat_sorted, W1, b1, W2, b2)
    g0 = _gather_half_sc(out_sorted, dest_flat, 0)
    g1 = _gather_half_sc(out_sorted, dest_flat, 1)
    logits0 = _combine_half(g0, p[:_N // 2], Wc, bc, None, 0)
    logits = _combine_half(g1, p[_N // 2:], Wc, bc, logits0, 1)
    return logits, p
